# baseline (speedup 1.0000x reference)
import functools

import jax
import jax.numpy as jnp
from jax.experimental import pallas as pl
from jax.experimental.pallas import tpu as pltpu

_E = 8
_D = 1024
_T = 2048


_NC = 1
_CW = _D // _NC


def _moe_body(x_ref, wg_ref, we_ref, be_ref, y_ref, z_ref, aux_ref, p_ref,
              xb_ref):
    e = pl.program_id(0)

    @pl.when(e == 0)
    def _gate():
        x = x_ref[...]
        logits = jax.lax.dot_general(
            x, wg_ref[...], (((1,), (1,)), ((), ())),
            preferred_element_type=jnp.float32)
        m = jnp.max(logits, axis=-1, keepdims=True)
        ex = jnp.exp(logits - m)
        s = jnp.sum(ex, axis=-1, keepdims=True)
        p = ex / s
        p_ref[...] = p
        log_z = m + jnp.log(s)
        z_ref[0, 0] = jnp.sum(log_z * log_z) / _T
        aux_ref[0, 0] = _E * jnp.mean(jnp.sum(p, axis=-1))
        y_ref[...] = jax.lax.dot_general(
            p, be_ref[:, 0, :], (((1,), (0,)), ((), ())),
            preferred_element_type=jnp.float32)
        xb_ref[...] = x.astype(jnp.bfloat16)

    lane = jax.lax.broadcasted_iota(jnp.int32, (_T, _E), 1)
    w = jnp.sum(jnp.where(lane == e, p_ref[...], 0.0), axis=-1,
                keepdims=True)
    for j in range(_NC):
        sl = slice(j * _CW, (j + 1) * _CW)
        wb = we_ref[0, sl, :].astype(jnp.bfloat16)
        h = jax.lax.dot_general(
            xb_ref[...], wb, (((1,), (1,)), ((), ())),
            preferred_element_type=jnp.float32)
        y_ref[:, sl] += w * h


@functools.partial(jax.jit, static_argnames=())
def kernel(x, Wg, We, be):
    orig_shape = x.shape
    xf = x.reshape(-1, x.shape[-1])
    y, z, aux = pl.pallas_call(
        _moe_body,
        grid=(_E,),
        in_specs=[
            pl.BlockSpec((_T, _D), lambda e: (0, 0)),
            pl.BlockSpec((_E, _D), lambda e: (0, 0)),
            pl.BlockSpec((1, _D, _D), lambda e: (e, 0, 0)),
            pl.BlockSpec((_E, 1, _D), lambda e: (0, 0, 0)),
        ],
        out_specs=[
            pl.BlockSpec((_T, _D), lambda e: (0, 0)),
            pl.BlockSpec(memory_space=pltpu.SMEM),
            pl.BlockSpec(memory_space=pltpu.SMEM),
        ],
        out_shape=[
            jax.ShapeDtypeStruct((_T, _D), jnp.float32),
            jax.ShapeDtypeStruct((1, 1), jnp.float32),
            jax.ShapeDtypeStruct((1, 1), jnp.float32),
        ],
        scratch_shapes=[pltpu.VMEM((_T, _E), jnp.float32),
                        pltpu.VMEM((_T, _D), jnp.bfloat16)],
        compiler_params=pltpu.CompilerParams(
            dimension_semantics=("arbitrary",)),
    )(xf, Wg, We, be.reshape(_E, 1, _D))
    return (y.reshape(orig_shape), z[0, 0], aux[0, 0])


# speedup vs baseline: 1.0094x; 1.0094x over previous
import functools

import jax
import jax.numpy as jnp
from jax.experimental import pallas as pl
from jax.experimental.pallas import tpu as pltpu

_E = 8
_D = 1024
_T = 2048


_EB = 2
_NS = _E // _EB


def _moe_body(x_ref, wg_ref, we_ref, be_ref, y_ref, z_ref, aux_ref, p_ref,
              xb_ref):
    e = pl.program_id(0)

    @pl.when(e == 0)
    def _gate():
        x = x_ref[...]
        logits = jax.lax.dot_general(
            x, wg_ref[...], (((1,), (1,)), ((), ())),
            preferred_element_type=jnp.float32)
        m = jnp.max(logits, axis=-1, keepdims=True)
        ex = jnp.exp(logits - m)
        s = jnp.sum(ex, axis=-1, keepdims=True)
        p = ex / s
        p_ref[...] = p
        log_z = m + jnp.log(s)
        z_ref[0, 0] = jnp.sum(log_z * log_z) / _T
        aux_ref[0, 0] = _E * jnp.mean(jnp.sum(p, axis=-1))
        y_ref[...] = jax.lax.dot_general(
            p, be_ref[:, 0, :], (((1,), (0,)), ((), ())),
            preferred_element_type=jnp.float32)
        xb_ref[...] = x.astype(jnp.bfloat16)

    lane = jax.lax.broadcasted_iota(jnp.int32, (_T, _E), 1)
    for q in range(_EB):
        w = jnp.sum(jnp.where(lane == e * _EB + q, p_ref[...], 0.0),
                    axis=-1, keepdims=True)
        wb = we_ref[q].astype(jnp.bfloat16)
        h = jax.lax.dot_general(
            xb_ref[...], wb, (((1,), (1,)), ((), ())),
            preferred_element_type=jnp.float32)
        y_ref[...] += w * h


@functools.partial(jax.jit, static_argnames=())
def kernel(x, Wg, We, be):
    orig_shape = x.shape
    xf = x.reshape(-1, x.shape[-1])
    y, z, aux = pl.pallas_call(
        _moe_body,
        grid=(_NS,),
        in_specs=[
            pl.BlockSpec((_T, _D), lambda e: (0, 0)),
            pl.BlockSpec((_E, _D), lambda e: (0, 0)),
            pl.BlockSpec((_EB, _D, _D), lambda e: (e, 0, 0)),
            pl.BlockSpec((_E, 1, _D), lambda e: (0, 0, 0)),
        ],
        out_specs=[
            pl.BlockSpec((_T, _D), lambda e: (0, 0)),
            pl.BlockSpec(memory_space=pltpu.SMEM),
            pl.BlockSpec(memory_space=pltpu.SMEM),
        ],
        out_shape=[
            jax.ShapeDtypeStruct((_T, _D), jnp.float32),
            jax.ShapeDtypeStruct((1, 1), jnp.float32),
            jax.ShapeDtypeStruct((1, 1), jnp.float32),
        ],
        scratch_shapes=[pltpu.VMEM((_T, _E), jnp.float32),
                        pltpu.VMEM((_T, _D), jnp.bfloat16)],
        compiler_params=pltpu.CompilerParams(
            dimension_semantics=("arbitrary",)),
    )(xf, Wg, We, be.reshape(_E, 1, _D))
    return (y.reshape(orig_shape), z[0, 0], aux[0, 0])
